# baseline (speedup 1.0000x reference)
import functools
import math

import jax
import jax.numpy as jnp
import numpy as np
from jax import lax
from jax.experimental import pallas as pl
from jax.experimental.pallas import tpu as pltpu
from jax.experimental.pallas import tpu_sc as plsc

ALPHA = 0.1
THETA = 0.5
CHUNK = 128
SK = 2
PAD_ROWS = 112


def _sc_info():
    try:
        info = plsc.get_sparse_core_info()
        return info.num_cores, info.num_subcores
    except Exception:
        return 2, 16


@functools.lru_cache(maxsize=None)
def _make_segment_sum(N, D, n_chunks):
    NC, NS = _sc_info()
    NW = NC * NS
    loop_iters = n_chunks // NW
    n_full = N // CHUNK
    rem = N - n_full * CHUNK
    N_acc = N + PAD_ROWS
    mesh = plsc.VectorSubcoreMesh(core_axis_name="c", subcore_axis_name="s")

    scr = (
        [pltpu.VMEM((CHUNK,), jnp.int32) for _ in range(2 * SK)]
        + [pltpu.VMEM((CHUNK, D), jnp.float32) for _ in range(SK)]
        + [pltpu.VMEM_SHARED((N_acc, D), jnp.float32)]
        + [pltpu.SemaphoreType.DMA for _ in range(3 * SK)]
    )

    @functools.partial(
        pl.kernel,
        mesh=mesh,
        out_type=jax.ShapeDtypeStruct((NC, N, D), jnp.float32),
        scratch_types=scr,
    )
    def seg(h_hbm, src_hbm, dst_hbm, zeros_hbm, out_hbm, *rest):
        sb = rest[:SK]
        db = rest[SK:2 * SK]
        rows = rest[2 * SK:3 * SK]
        acc = rest[3 * SK]
        sems = rest[3 * SK + 1:]
        c = lax.axis_index("c")
        s = lax.axis_index("s")
        w = s * NC + c

        def for_each_row_block(fn, nf, rm):
            iters = math.ceil((nf + (1 if rm else 0)) / NS)
            for i in range(iters):
                b = s + NS * i

                @pl.when(b < nf)
                def _():
                    fn(b * CHUNK, CHUNK)

                if rm:

                    @pl.when(b == nf)
                    def _():
                        fn(nf * CHUNK, rm)

        for_each_row_block(lambda base, sz: pltpu.sync_copy(
            zeros_hbm.at[pl.ds(0, sz)], acc.at[pl.ds(base, sz)]),
            N_acc // CHUNK, N_acc % CHUNK)
        plsc.subcore_barrier()

        def body(i, carry):
            base = (w + NW * i) * CHUNK
            pltpu.sync_copy(src_hbm.at[pl.ds(base, CHUNK)], sb[0])
            pltpu.sync_copy(dst_hbm.at[pl.ds(base, CHUNK)], db[0])
            pltpu.async_copy(h_hbm.at[sb[0]], rows[0], sems[0]).wait()
            pltpu.sync_copy(rows[0], acc.at[db[0]], add=True)
            return carry

        lax.fori_loop(0, loop_iters, body, None)
        plsc.subcore_barrier()
        for_each_row_block(lambda base, sz: pltpu.sync_copy(
            acc.at[pl.ds(base, sz)], out_hbm.at[c, pl.ds(base, sz)]),
            n_full, rem)

    return seg


def _mm_relu_body(x_ref, w_ref, b_ref, o_ref):
    y = jnp.dot(x_ref[...], w_ref[...], preferred_element_type=jnp.float32)
    o_ref[...] = jnp.maximum(y + b_ref[...], 0.0)


def _combine_body(p0_ref, p1_ref, x0_ref, w_ref, o_ref, *, beta):
    z = (1.0 - ALPHA) * (p0_ref[...] + p1_ref[...]) + ALPHA * x0_ref[...]
    y = (1.0 - beta) * z + beta * jnp.dot(z, w_ref[...], preferred_element_type=jnp.float32)
    o_ref[...] = jnp.maximum(y, 0.0)


def _final_body(h_ref, w_ref, b_ref, o_ref, *, C):
    logits = jnp.dot(h_ref[...], w_ref[...], preferred_element_type=jnp.float32) + b_ref[...]
    col = lax.broadcasted_iota(jnp.int32, logits.shape, 1)
    valid = col < C
    masked = jnp.where(valid, logits, -jnp.inf)
    m = jnp.max(masked, axis=1, keepdims=True)
    ex = jnp.where(valid, jnp.exp(masked - m), 0.0)
    lse = jnp.log(jnp.sum(ex, axis=1, keepdims=True)) + m
    o_ref[...] = logits - lse


def _tc_call(body, out_shape, *args):
    return pl.pallas_call(body, out_shape=out_shape)(*args)


def kernel(x, edge_index, W0, b0, Wc, W1, b1):
    N, D = x.shape
    H = W0.shape[1]
    C = W1.shape[1]
    L = Wc.shape[0]
    E = edge_index.shape[1]
    NC, NS = _sc_info()
    NW = NC * NS

    E_pad = math.ceil(E / (CHUNK * NW * SK)) * CHUNK * NW * SK
    n_chunks = E_pad // CHUNK
    src = edge_index[0].astype(jnp.int32)
    dst = edge_index[1].astype(jnp.int32)
    pad = E_pad - E
    src_p = jnp.concatenate([src, jnp.zeros((pad,), jnp.int32)])
    dst_p = jnp.concatenate(
        [dst, N + (jnp.arange(pad, dtype=jnp.int32) % PAD_ROWS)])
    zeros = jnp.zeros((CHUNK, H), jnp.float32)

    f32 = jnp.float32
    h = _tc_call(_mm_relu_body, jax.ShapeDtypeStruct((N, H), f32),
                 x, W0, b0.reshape(1, H))
    x0 = h
    seg = _make_segment_sum(N, H, n_chunks)
    for l in range(L):
        beta = float(np.log(THETA / (l + 1) + 1.0))
        partials = seg(h, src_p, dst_p, zeros)
        h = _tc_call(functools.partial(_combine_body, beta=beta),
                     jax.ShapeDtypeStruct((N, H), f32),
                     partials[0], partials[1], x0, Wc[l])

    Wp = jnp.zeros((H, 128), f32).at[:, :C].set(W1)
    bp = jnp.zeros((1, 128), f32).at[0, :C].set(b1)
    out = _tc_call(functools.partial(_final_body, C=C),
                   jax.ShapeDtypeStruct((N, 128), f32),
                   h, Wp, bp)
    return out[:, :C]


# speedup vs baseline: 1.9690x; 1.9690x over previous
import functools
import math

import jax
import jax.numpy as jnp
import numpy as np
from jax import lax
from jax.experimental import pallas as pl
from jax.experimental.pallas import tpu as pltpu
from jax.experimental.pallas import tpu_sc as plsc

ALPHA = 0.1
THETA = 0.5
CHUNK = 128
SK = 2


def _sc_info():
    try:
        info = plsc.get_sparse_core_info()
        return info.num_cores, info.num_subcores
    except Exception:
        return 2, 16


@functools.lru_cache(maxsize=None)
def _make_segment_sum(N, D, n_chunks):
    NC, NS = _sc_info()
    NW = NC * NS
    loop_iters = n_chunks // NW
    n_full = N // CHUNK
    rem = N - n_full * CHUNK
    mesh = plsc.VectorSubcoreMesh(core_axis_name="c", subcore_axis_name="s")

    scr = (
        [pltpu.VMEM((CHUNK,), jnp.int32) for _ in range(2 * SK)]
        + [pltpu.VMEM((CHUNK, D), jnp.float32) for _ in range(SK)]
        + [pltpu.VMEM_SHARED((N, D), jnp.float32)]
        + [pltpu.SemaphoreType.DMA for _ in range(3 * SK)]
    )

    @functools.partial(
        pl.kernel,
        mesh=mesh,
        out_type=jax.ShapeDtypeStruct((NC, N, D), jnp.float32),
        scratch_types=scr,
    )
    def seg(h_hbm, src_hbm, dst_hbm, zeros_hbm, out_hbm, *rest):
        sb = rest[:SK]
        db = rest[SK:2 * SK]
        rows = rest[2 * SK:3 * SK]
        acc = rest[3 * SK]
        sems = rest[3 * SK + 1:]
        c = lax.axis_index("c")
        s = lax.axis_index("s")
        w = s * NC + c

        def for_each_row_block(fn, nf, rm):
            iters = math.ceil((nf + (1 if rm else 0)) / NS)
            for i in range(iters):
                b = s + NS * i

                @pl.when(b < nf)
                def _():
                    fn(b * CHUNK, CHUNK)

                if rm:

                    @pl.when(b == nf)
                    def _():
                        fn(nf * CHUNK, rm)

        for_each_row_block(lambda base, sz: pltpu.sync_copy(
            zeros_hbm.at[pl.ds(0, sz)], acc.at[pl.ds(base, sz)]),
            n_full, rem)
        plsc.subcore_barrier()

        def body(i, carry):
            base = (w + NW * i) * CHUNK
            pltpu.sync_copy(src_hbm.at[pl.ds(base, CHUNK)], sb[0])
            pltpu.sync_copy(dst_hbm.at[pl.ds(base, CHUNK)], db[0])
            pltpu.async_copy(h_hbm.at[sb[0]], rows[0], sems[0]).wait()
            pltpu.sync_copy(rows[0], acc.at[db[0]], add=True)
            return carry

        lax.fori_loop(0, loop_iters, body, None)
        plsc.subcore_barrier()
        for_each_row_block(lambda base, sz: pltpu.sync_copy(
            acc.at[pl.ds(base, sz)], out_hbm.at[c, pl.ds(base, sz)]),
            n_full, rem)

    return seg


def _mm_relu_body(x_ref, w_ref, b_ref, o_ref, *, zpad):
    y = jnp.dot(x_ref[...], w_ref[...], preferred_element_type=jnp.float32)
    n = x_ref.shape[0]
    o_ref[pl.ds(0, n), :] = jnp.maximum(y + b_ref[...], 0.0)
    o_ref[pl.ds(n, zpad), :] = jnp.zeros((zpad, o_ref.shape[1]), jnp.float32)


def _combine_body(p0_ref, p1_ref, x0_ref, w_ref, o_ref, *, beta):
    z = (1.0 - ALPHA) * (p0_ref[...] + p1_ref[...]) + ALPHA * x0_ref[...]
    y = (1.0 - beta) * z + beta * jnp.dot(z, w_ref[...], preferred_element_type=jnp.float32)
    o_ref[...] = jnp.maximum(y, 0.0)


def _final_body(h_ref, w_ref, b_ref, o_ref, *, C):
    logits = jnp.dot(h_ref[...], w_ref[...], preferred_element_type=jnp.float32) + b_ref[...]
    col = lax.broadcasted_iota(jnp.int32, logits.shape, 1)
    valid = col < C
    masked = jnp.where(valid, logits, -jnp.inf)
    m = jnp.max(masked, axis=1, keepdims=True)
    ex = jnp.where(valid, jnp.exp(masked - m), 0.0)
    lse = jnp.log(jnp.sum(ex, axis=1, keepdims=True)) + m
    o_ref[...] = logits - lse


def _tc_call(body, out_shape, *args):
    return pl.pallas_call(body, out_shape=out_shape)(*args)


def kernel(x, edge_index, W0, b0, Wc, W1, b1):
    N, D = x.shape
    H = W0.shape[1]
    C = W1.shape[1]
    L = Wc.shape[0]
    E = edge_index.shape[1]
    NC, NS = _sc_info()
    NW = NC * NS

    ZPAD = 8
    Np = N + ZPAD
    E_pad = math.ceil(E / (CHUNK * NW)) * CHUNK * NW
    n_chunks = E_pad // CHUNK
    src = edge_index[0].astype(jnp.int32)
    dst = edge_index[1].astype(jnp.int32)
    pad = E_pad - E
    src_p = jnp.concatenate(
        [src, N + (jnp.arange(pad, dtype=jnp.int32) % ZPAD)])
    dst_p = jnp.concatenate(
        [dst, jnp.arange(pad, dtype=jnp.int32) % N])
    zeros = jnp.zeros((CHUNK, H), jnp.float32)

    f32 = jnp.float32
    h = _tc_call(functools.partial(_mm_relu_body, zpad=ZPAD),
                 jax.ShapeDtypeStruct((Np, H), f32),
                 x, W0, b0.reshape(1, H))
    x0 = h
    seg = _make_segment_sum(Np, H, n_chunks)
    for l in range(L):
        beta = float(np.log(THETA / (l + 1) + 1.0))
        partials = seg(h, src_p, dst_p, zeros)
        h = _tc_call(functools.partial(_combine_body, beta=beta),
                     jax.ShapeDtypeStruct((Np, H), f32),
                     partials[0], partials[1], x0, Wc[l])

    Wp = jnp.zeros((H, 128), f32).at[:, :C].set(W1)
    bp = jnp.zeros((1, 128), f32).at[0, :C].set(b1)
    out = _tc_call(functools.partial(_final_body, C=C),
                   jax.ShapeDtypeStruct((Np, 128), f32),
                   h, Wp, bp)
    return out[:N, :C]


# speedup vs baseline: 2.4470x; 1.2427x over previous
import functools
import math

import jax
import jax.numpy as jnp
import numpy as np
from jax import lax
from jax.experimental import pallas as pl
from jax.experimental.pallas import tpu as pltpu
from jax.experimental.pallas import tpu_sc as plsc

ALPHA = 0.1
THETA = 0.5
CHUNK = 128
SK = 2


def _sc_info():
    try:
        info = plsc.get_sparse_core_info()
        return info.num_cores, info.num_subcores
    except Exception:
        return 2, 16


@functools.lru_cache(maxsize=None)
def _make_segment_sum(N, D, n_chunks):
    NC, NS = _sc_info()
    NW = NC * NS
    loop_iters = n_chunks // (NW * SK)
    n_full = N // CHUNK
    rem = N - n_full * CHUNK
    mesh = plsc.VectorSubcoreMesh(core_axis_name="c", subcore_axis_name="s")

    scr = (
        [pltpu.VMEM((CHUNK,), jnp.int32) for _ in range(2 * SK)]
        + [pltpu.VMEM((CHUNK, D), jnp.float32) for _ in range(SK)]
        + [pltpu.VMEM_SHARED((N, D), jnp.float32)]
        + [pltpu.SemaphoreType.DMA for _ in range(3 * SK)]
    )

    @functools.partial(
        pl.kernel,
        mesh=mesh,
        out_type=jax.ShapeDtypeStruct((NC, N, D), jnp.float32),
        scratch_types=scr,
    )
    def seg(h_hbm, src_hbm, dst_hbm, zeros_hbm, out_hbm, *rest):
        sb = rest[:SK]
        db = rest[SK:2 * SK]
        rows = rest[2 * SK:3 * SK]
        acc = rest[3 * SK]
        sems = rest[3 * SK + 1:]
        c = lax.axis_index("c")
        s = lax.axis_index("s")
        w = s * NC + c

        def for_each_row_block(fn, nf, rm):
            iters = math.ceil((nf + (1 if rm else 0)) / NS)
            for i in range(iters):
                b = s + NS * i

                @pl.when(b < nf)
                def _():
                    fn(b * CHUNK, CHUNK)

                if rm:

                    @pl.when(b == nf)
                    def _():
                        fn(nf * CHUNK, rm)

        for_each_row_block(lambda base, sz: pltpu.sync_copy(
            zeros_hbm.at[pl.ds(0, sz)], acc.at[pl.ds(base, sz)]),
            n_full, rem)
        plsc.subcore_barrier()

        def body(tt, carry):
            idx_cps = []
            for j in range(SK):
                base = (w + NW * (SK * tt + j)) * CHUNK
                idx_cps.append(pltpu.async_copy(
                    src_hbm.at[pl.ds(base, CHUNK)], sb[j], sems[3 * j]))
                idx_cps.append(pltpu.async_copy(
                    dst_hbm.at[pl.ds(base, CHUNK)], db[j], sems[3 * j + 1]))
            gathers = []
            for j in range(SK):
                idx_cps[2 * j].wait()
                gathers.append(pltpu.async_copy(
                    h_hbm.at[sb[j]], rows[j], sems[3 * j + 2]))
            for j in range(SK):
                idx_cps[2 * j + 1].wait()
                gathers[j].wait()
                pltpu.sync_copy(rows[j], acc.at[db[j]], add=True)
            return carry

        lax.fori_loop(0, loop_iters, body, None)
        plsc.subcore_barrier()
        for_each_row_block(lambda base, sz: pltpu.sync_copy(
            acc.at[pl.ds(base, sz)], out_hbm.at[c, pl.ds(base, sz)]),
            n_full, rem)

    return seg


def _mm_relu_body(x_ref, w_ref, b_ref, o_ref, *, zpad):
    y = jnp.dot(x_ref[...], w_ref[...], preferred_element_type=jnp.float32)
    n = x_ref.shape[0]
    o_ref[pl.ds(0, n), :] = jnp.maximum(y + b_ref[...], 0.0)
    o_ref[pl.ds(n, zpad), :] = jnp.zeros((zpad, o_ref.shape[1]), jnp.float32)


def _combine_body(p0_ref, p1_ref, x0_ref, w_ref, o_ref, *, beta):
    z = (1.0 - ALPHA) * (p0_ref[...] + p1_ref[...]) + ALPHA * x0_ref[...]
    y = (1.0 - beta) * z + beta * jnp.dot(z, w_ref[...], preferred_element_type=jnp.float32)
    o_ref[...] = jnp.maximum(y, 0.0)


def _final_body(h_ref, w_ref, b_ref, o_ref, *, C):
    logits = jnp.dot(h_ref[...], w_ref[...], preferred_element_type=jnp.float32) + b_ref[...]
    col = lax.broadcasted_iota(jnp.int32, logits.shape, 1)
    valid = col < C
    masked = jnp.where(valid, logits, -jnp.inf)
    m = jnp.max(masked, axis=1, keepdims=True)
    ex = jnp.where(valid, jnp.exp(masked - m), 0.0)
    lse = jnp.log(jnp.sum(ex, axis=1, keepdims=True)) + m
    o_ref[...] = logits - lse


def _tc_call(body, out_shape, *args):
    return pl.pallas_call(body, out_shape=out_shape)(*args)


def kernel(x, edge_index, W0, b0, Wc, W1, b1):
    N, D = x.shape
    H = W0.shape[1]
    C = W1.shape[1]
    L = Wc.shape[0]
    E = edge_index.shape[1]
    NC, NS = _sc_info()
    NW = NC * NS

    ZPAD = 8
    Np = N + ZPAD
    E_pad = math.ceil(E / (CHUNK * NW * SK)) * CHUNK * NW * SK
    n_chunks = E_pad // CHUNK
    src = edge_index[0].astype(jnp.int32)
    dst = edge_index[1].astype(jnp.int32)
    pad = E_pad - E
    src_p = jnp.concatenate(
        [src, N + (jnp.arange(pad, dtype=jnp.int32) % ZPAD)])
    dst_p = jnp.concatenate(
        [dst, jnp.arange(pad, dtype=jnp.int32) % N])
    zeros = jnp.zeros((CHUNK, H), jnp.float32)

    f32 = jnp.float32
    h = _tc_call(functools.partial(_mm_relu_body, zpad=ZPAD),
                 jax.ShapeDtypeStruct((Np, H), f32),
                 x, W0, b0.reshape(1, H))
    x0 = h
    seg = _make_segment_sum(Np, H, n_chunks)
    for l in range(L):
        beta = float(np.log(THETA / (l + 1) + 1.0))
        partials = seg(h, src_p, dst_p, zeros)
        h = _tc_call(functools.partial(_combine_body, beta=beta),
                     jax.ShapeDtypeStruct((Np, H), f32),
                     partials[0], partials[1], x0, Wc[l])

    Wp = jnp.zeros((H, 128), f32).at[:, :C].set(W1)
    bp = jnp.zeros((1, 128), f32).at[0, :C].set(b1)
    out = _tc_call(functools.partial(_final_body, C=C),
                   jax.ShapeDtypeStruct((Np, 128), f32),
                   h, Wp, bp)
    return out[:N, :C]


# speedup vs baseline: 3.3118x; 1.3534x over previous
import functools
import math

import jax
import jax.numpy as jnp
import numpy as np
from jax import lax
from jax.experimental import pallas as pl
from jax.experimental.pallas import tpu as pltpu
from jax.experimental.pallas import tpu_sc as plsc

ALPHA = 0.1
THETA = 0.5
CHUNK = 128
NBUF = 2
NSRC = 4


def _sc_info():
    try:
        info = plsc.get_sparse_core_info()
        return info.num_cores, info.num_subcores
    except Exception:
        return 2, 16


@functools.lru_cache(maxsize=None)
def _make_segment_sum(N, D, n_chunks):
    NC, NS = _sc_info()
    NW = NC * NS
    loop_iters = n_chunks // (NW * NSRC)
    n_full = N // CHUNK
    rem = N - n_full * CHUNK
    mesh = plsc.VectorSubcoreMesh(core_axis_name="c", subcore_axis_name="s")

    scr = (
        [pltpu.VMEM((CHUNK,), jnp.int32) for _ in range(NSRC)]
        + [pltpu.VMEM((CHUNK,), jnp.int32) for _ in range(NBUF)]
        + [pltpu.VMEM((CHUNK, D), jnp.float32) for _ in range(NBUF)]
        + [pltpu.VMEM_SHARED((N, D), jnp.float32)]
        + [pltpu.SemaphoreType.DMA for _ in range(NSRC)]
        + [pltpu.SemaphoreType.DMA for _ in range(NBUF)]
        + [pltpu.SemaphoreType.DMA for _ in range(NBUF)]
    )

    @functools.partial(
        pl.kernel,
        mesh=mesh,
        out_type=jax.ShapeDtypeStruct((NC, N, D), jnp.float32),
        scratch_types=scr,
    )
    def seg(h_hbm, src_hbm, dst_hbm, zeros_hbm, out_hbm, *rest):
        sb = rest[:NSRC]
        db = rest[NSRC:NSRC + NBUF]
        rows = rest[NSRC + NBUF:NSRC + 2 * NBUF]
        acc = rest[NSRC + 2 * NBUF]
        ssem = rest[NSRC + 2 * NBUF + 1:2 * NSRC + 2 * NBUF + 1]
        dsem = rest[2 * NSRC + 2 * NBUF + 1:2 * NSRC + 3 * NBUF + 1]
        gsem = rest[2 * NSRC + 3 * NBUF + 1:]
        c = lax.axis_index("c")
        s = lax.axis_index("s")
        w = s * NC + c

        def for_each_row_block(fn, nf, rm):
            iters = math.ceil((nf + (1 if rm else 0)) / NS)
            for i in range(iters):
                b = s + NS * i

                @pl.when(b < nf)
                def _():
                    fn(b * CHUNK, CHUNK)

                if rm:

                    @pl.when(b == nf)
                    def _():
                        fn(nf * CHUNK, rm)

        for_each_row_block(lambda base, sz: pltpu.sync_copy(
            zeros_hbm.at[pl.ds(0, sz)], acc.at[pl.ds(base, sz)]),
            n_full, rem)
        plsc.subcore_barrier()

        def cbase(i):
            return (w + NW * i) * CHUNK

        def prefetch_src(i, slot):
            pltpu.async_copy(src_hbm.at[pl.ds(cbase(i), CHUNK)], sb[slot],
                             ssem[slot])

        def wait_src(i, slot):
            pltpu.make_async_copy(src_hbm.at[pl.ds(cbase(i), CHUNK)],
                                  sb[slot], ssem[slot]).wait()

        def prefetch_dst(i, b):
            pltpu.async_copy(dst_hbm.at[pl.ds(cbase(i), CHUNK)], db[b],
                             dsem[b])

        def wait_dst(i, b):
            pltpu.make_async_copy(dst_hbm.at[pl.ds(cbase(i), CHUNK)],
                                  db[b], dsem[b]).wait()

        def gather(slot, b):
            pltpu.async_copy(h_hbm.at[sb[slot]], rows[b], gsem[b])

        def wait_gather(slot, b):
            pltpu.make_async_copy(h_hbm.at[sb[slot]], rows[b],
                                  gsem[b]).wait()

        for j in range(NSRC):
            prefetch_src(j, j)
        for b in range(NBUF):
            prefetch_dst(b, b)
        for b in range(NBUF):
            wait_src(b, b)
            gather(b, b)

        total = loop_iters * NSRC
        outer = loop_iters

        def body(g, carry):
            for b in range(NSRC):
                i = g * NSRC + b
                rb = b % NBUF
                wait_gather(b, rb)
                wait_dst(i, rb)
                pltpu.sync_copy(rows[rb], acc.at[db[rb]], add=True)

                @pl.when(g + 1 < outer)
                def _():
                    prefetch_src(i + NSRC, b)

                nb = (b + NBUF) % NSRC

                @pl.when(i + NBUF < total)
                def _():
                    prefetch_dst(i + NBUF, rb)
                    wait_src(i + NBUF, nb)
                    gather(nb, rb)
            return carry

        lax.fori_loop(0, outer, body, None)
        plsc.subcore_barrier()
        for_each_row_block(lambda base, sz: pltpu.sync_copy(
            acc.at[pl.ds(base, sz)], out_hbm.at[c, pl.ds(base, sz)]),
            n_full, rem)

    return seg


def _mm_relu_body(x_ref, w_ref, b_ref, o_ref, *, zpad):
    y = jnp.dot(x_ref[...], w_ref[...], preferred_element_type=jnp.float32)
    n = x_ref.shape[0]
    o_ref[pl.ds(0, n), :] = jnp.maximum(y + b_ref[...], 0.0)
    o_ref[pl.ds(n, zpad), :] = jnp.zeros((zpad, o_ref.shape[1]), jnp.float32)


def _combine_body(p0_ref, p1_ref, x0_ref, w_ref, o_ref, *, beta):
    z = (1.0 - ALPHA) * (p0_ref[...] + p1_ref[...]) + ALPHA * x0_ref[...]
    y = (1.0 - beta) * z + beta * jnp.dot(z, w_ref[...], preferred_element_type=jnp.float32)
    o_ref[...] = jnp.maximum(y, 0.0)


def _final_body(h_ref, w_ref, b_ref, o_ref, *, C):
    logits = jnp.dot(h_ref[...], w_ref[...], preferred_element_type=jnp.float32) + b_ref[...]
    col = lax.broadcasted_iota(jnp.int32, logits.shape, 1)
    valid = col < C
    masked = jnp.where(valid, logits, -jnp.inf)
    m = jnp.max(masked, axis=1, keepdims=True)
    ex = jnp.where(valid, jnp.exp(masked - m), 0.0)
    lse = jnp.log(jnp.sum(ex, axis=1, keepdims=True)) + m
    o_ref[...] = logits - lse


def _tc_call(body, out_shape, *args):
    return pl.pallas_call(body, out_shape=out_shape)(*args)


def kernel(x, edge_index, W0, b0, Wc, W1, b1):
    N, D = x.shape
    H = W0.shape[1]
    C = W1.shape[1]
    L = Wc.shape[0]
    E = edge_index.shape[1]
    NC, NS = _sc_info()
    NW = NC * NS

    ZPAD = 8
    Np = N + ZPAD
    E_pad = math.ceil(E / (CHUNK * NW * NSRC)) * CHUNK * NW * NSRC
    n_chunks = E_pad // CHUNK
    src = edge_index[0].astype(jnp.int32)
    dst = edge_index[1].astype(jnp.int32)
    pad = E_pad - E
    src_p = jnp.concatenate(
        [src, N + (jnp.arange(pad, dtype=jnp.int32) % ZPAD)])
    dst_p = jnp.concatenate(
        [dst, jnp.arange(pad, dtype=jnp.int32) % N])
    zeros = jnp.zeros((CHUNK, H), jnp.float32)

    f32 = jnp.float32
    h = _tc_call(functools.partial(_mm_relu_body, zpad=ZPAD),
                 jax.ShapeDtypeStruct((Np, H), f32),
                 x, W0, b0.reshape(1, H))
    x0 = h
    seg = _make_segment_sum(Np, H, n_chunks)
    for l in range(L):
        beta = float(np.log(THETA / (l + 1) + 1.0))
        partials = seg(h, src_p, dst_p, zeros)
        h = _tc_call(functools.partial(_combine_body, beta=beta),
                     jax.ShapeDtypeStruct((Np, H), f32),
                     partials[0], partials[1], x0, Wc[l])

    Wp = jnp.zeros((H, 128), f32).at[:, :C].set(W1)
    bp = jnp.zeros((1, 128), f32).at[0, :C].set(b1)
    out = _tc_call(functools.partial(_final_body, C=C),
                   jax.ShapeDtypeStruct((Np, 128), f32),
                   h, Wp, bp)
    return out[:N, :C]
